# Initial kernel scaffold; baseline (speedup 1.0000x reference)
import jax
import jax.numpy as jnp
from jax.experimental import pallas as pl


def kernel(x, edge_index, batch, W0, b0, W1, as1, ad1, b1, W2, as2, ad2, b2, W3, as3, ad3, b3):
    raise NotImplementedError("write your pallas kernel here")


# speedup vs baseline: 14.4279x; 14.4279x over previous
import functools

import jax
import jax.numpy as jnp
from jax import lax
from jax.experimental import pallas as pl
from jax.experimental.pallas import tpu as pltpu
from jax.experimental.pallas import tpu_sc as plsc

N_NODES = 10000
NHAT = 10240
MB = 1024
N_BLKS = NHAT // MB
E_REAL = 330000
TILES = 16
K = 128
CHUNKS = 162
E_PAD = TILES * CHUNKS * K
ROWS_PER_TILE = NHAT // TILES


def _tc1_body(x_ref, w0_ref, b0_ref, w1_ref, avs_ref, avd_ref,
              xl0_ref, xl1_ref, asrc_ref, adst_ref):
    t = jnp.dot(x_ref[...], w0_ref[...], preferred_element_type=jnp.float32)
    t = t + b0_ref[...]
    xl = jnp.dot(t, w1_ref[...], preferred_element_type=jnp.float32)
    fh = xl.shape[1] // 2
    xl0_ref[...] = xl[:, :fh]
    xl1_ref[...] = xl[:, fh:]
    asrc_ref[...] = jnp.sum(xl * avs_ref[...], axis=1).reshape(8, 128)
    adst_ref[...] = jnp.sum(xl * avd_ref[...], axis=1).reshape(8, 128)


def _tc_layer1(x, w0, b0, w1, avs, avd):
    fo = w1.shape[1]
    fh = fo // 2
    return pl.pallas_call(
        _tc1_body,
        grid=(N_BLKS,),
        in_specs=[
            pl.BlockSpec((MB, 128), lambda i: (i, 0)),
            pl.BlockSpec((128, 128), lambda i: (0, 0)),
            pl.BlockSpec((1, 128), lambda i: (0, 0)),
            pl.BlockSpec((128, fo), lambda i: (0, 0)),
            pl.BlockSpec((1, fo), lambda i: (0, 0)),
            pl.BlockSpec((1, fo), lambda i: (0, 0)),
        ],
        out_specs=[
            pl.BlockSpec((MB, fh), lambda i: (i, 0)),
            pl.BlockSpec((MB, fh), lambda i: (i, 0)),
            pl.BlockSpec((8, 128), lambda i: (i, 0)),
            pl.BlockSpec((8, 128), lambda i: (i, 0)),
        ],
        out_shape=[
            jax.ShapeDtypeStruct((NHAT, fh), jnp.float32),
            jax.ShapeDtypeStruct((NHAT, fh), jnp.float32),
            jax.ShapeDtypeStruct((NHAT // 128, 128), jnp.float32),
            jax.ShapeDtypeStruct((NHAT // 128, 128), jnp.float32),
        ],
    )(x, w0, b0.reshape(1, -1), w1, avs.reshape(1, -1), avd.reshape(1, -1))


def _tcmid_body(u0_ref, u1_ref, bp_ref, w_ref, avs_ref, avd_ref,
                xl0_ref, xl1_ref, asrc_ref, adst_ref):
    fhin = u0_ref.shape[1]
    b = bp_ref[...]
    h0 = jnp.maximum(u0_ref[...] + b[:, :fhin], 0.0)
    h1 = jnp.maximum(u1_ref[...] + b[:, fhin:], 0.0)
    w = w_ref[...]
    xl = (jnp.dot(h0, w[:fhin, :], preferred_element_type=jnp.float32)
          + jnp.dot(h1, w[fhin:, :], preferred_element_type=jnp.float32))
    fh = xl.shape[1] // 2
    xl0_ref[...] = xl[:, :fh]
    xl1_ref[...] = xl[:, fh:]
    asrc_ref[...] = jnp.sum(xl * avs_ref[...], axis=1).reshape(8, 128)
    adst_ref[...] = jnp.sum(xl * avd_ref[...], axis=1).reshape(8, 128)


def _tc_mid(u_flat, bp, w, avs, avd):
    fhin = u_flat.shape[1]
    fin, fo = w.shape
    fh = fo // 2
    return pl.pallas_call(
        _tcmid_body,
        grid=(N_BLKS,),
        in_specs=[
            pl.BlockSpec((MB, fhin), lambda i: (i, 0)),
            pl.BlockSpec((MB, fhin), lambda i: (i + N_BLKS, 0)),
            pl.BlockSpec((1, fin), lambda i: (0, 0)),
            pl.BlockSpec((fin, fo), lambda i: (0, 0)),
            pl.BlockSpec((1, fo), lambda i: (0, 0)),
            pl.BlockSpec((1, fo), lambda i: (0, 0)),
        ],
        out_specs=[
            pl.BlockSpec((MB, fh), lambda i: (i, 0)),
            pl.BlockSpec((MB, fh), lambda i: (i, 0)),
            pl.BlockSpec((8, 128), lambda i: (i, 0)),
            pl.BlockSpec((8, 128), lambda i: (i, 0)),
        ],
        out_shape=[
            jax.ShapeDtypeStruct((NHAT, fh), jnp.float32),
            jax.ShapeDtypeStruct((NHAT, fh), jnp.float32),
            jax.ShapeDtypeStruct((NHAT // 128, 128), jnp.float32),
            jax.ShapeDtypeStruct((NHAT // 128, 128), jnp.float32),
        ],
    )(u_flat, u_flat, bp.reshape(1, -1), w, avs.reshape(1, -1), avd.reshape(1, -1))


def _pool_body(u0_ref, u1_ref, b3_ref, batch_ref, out_ref, sums, cnts):
    i = pl.program_id(0)

    @pl.when(i == 0)
    def _():
        sums[...] = jnp.zeros_like(sums)
        cnts[...] = jnp.zeros_like(cnts)

    b = b3_ref[...]
    h = jnp.maximum(jnp.concatenate([u0_ref[...], u1_ref[...]], axis=1) + b, 0.0)
    groups = lax.broadcasted_iota(jnp.int32, (MB, 16), 1)
    oh = (batch_ref[...] == groups).astype(jnp.float32)
    dn = (((0,), (0,)), ((), ()))
    sums[...] += lax.dot_general(oh, h, dn, preferred_element_type=jnp.float32)
    cnts[...] += lax.dot_general(oh, jnp.ones_like(h), dn,
                                 preferred_element_type=jnp.float32)

    @pl.when(i == N_BLKS - 1)
    def _():
        out_ref[...] = sums[...] / jnp.maximum(cnts[...], 1.0)


def _pool(u_flat, b3, batch_col):
    fhin = u_flat.shape[1]
    return pl.pallas_call(
        _pool_body,
        grid=(N_BLKS,),
        in_specs=[
            pl.BlockSpec((MB, fhin), lambda i: (i, 0)),
            pl.BlockSpec((MB, fhin), lambda i: (i + N_BLKS, 0)),
            pl.BlockSpec((1, 128), lambda i: (0, 0)),
            pl.BlockSpec((MB, 1), lambda i: (i, 0)),
        ],
        out_specs=pl.BlockSpec((16, 128), lambda i: (0, 0)),
        out_shape=jax.ShapeDtypeStruct((16, 128), jnp.float32),
        scratch_shapes=[
            pltpu.VMEM((16, 128), jnp.float32),
            pltpu.VMEM((16, 128), jnp.float32),
        ],
    )(u_flat, u_flat, b3.reshape(1, -1), batch_col)


@functools.lru_cache(maxsize=None)
def _make_sc_gat(fh):
    fv = fh // 16
    mesh = plsc.VectorSubcoreMesh(core_axis_name="c", subcore_axis_name="s")

    @functools.partial(
        pl.kernel,
        out_type=jax.ShapeDtypeStruct((2 * NHAT, fh), jnp.float32),
        mesh=mesh,
        compiler_params=pltpu.CompilerParams(needs_layout_passes=False,
                                             use_tc_tiling_on_sc=False),
        scratch_types=[
            pltpu.VMEM((NHAT // 128, 128), jnp.float32),
            pltpu.VMEM((NHAT // 128, 128), jnp.float32),
            pltpu.VMEM((K,), jnp.int32),
            pltpu.VMEM((K,), jnp.int32),
            pltpu.VMEM((K,), jnp.int32),
            pltpu.VMEM((K,), jnp.float32),
            pltpu.VMEM((K, fh), jnp.float32),
            pltpu.VMEM((ROWS_PER_TILE,), jnp.float32),
            pltpu.VMEM_SHARED((NHAT, fh), jnp.float32),
            pltpu.VMEM_SHARED((NHAT,), jnp.float32),
            pltpu.SemaphoreType.DMA,
        ],
    )
    def gat(xl_hbm, srcs_hbm, dsts_hbm, asrc_hbm, adst_hbm, out_hbm,
            asrc_v, adst_v, idx_sr, idx_adj, idx_d, w_v, gbuf, dbuf,
            out_sh, den_sh, sem):
        cid = lax.axis_index("c")
        tid = lax.axis_index("s")
        row0 = tid * ROWS_PER_TILE

        def zrow(r, c):
            for j in range(fv):
                gbuf[r, pl.ds(j * 16, 16)] = jnp.zeros((16,), jnp.float32)
            return c
        lax.fori_loop(0, K, zrow, 0)
        for z in range(ROWS_PER_TILE // K):
            pltpu.sync_copy(gbuf, out_sh.at[pl.ds(row0 + z * K, K)])

        def zden(j, c):
            dbuf[pl.ds(j * 16, 16)] = jnp.zeros((16,), jnp.float32)
            return c
        lax.fori_loop(0, ROWS_PER_TILE // 16, zden, 0)
        pltpu.sync_copy(dbuf, den_sh.at[pl.ds(row0, ROWS_PER_TILE)])

        pltpu.sync_copy(asrc_hbm, asrc_v)
        pltpu.sync_copy(adst_hbm, adst_v)
        plsc.subcore_barrier()

        core_off = cid * NHAT

        def chunk(ch, c):
            base = (tid * CHUNKS + ch) * K
            pltpu.sync_copy(srcs_hbm.at[pl.ds(base, K)], idx_sr)
            pltpu.sync_copy(dsts_hbm.at[pl.ds(base, K)], idx_d)
            for v in range(K // 16):
                sv = idx_sr[pl.ds(v * 16, 16)]
                dv = idx_d[pl.ds(v * 16, 16)]
                a_s = plsc.load_gather(asrc_v, [sv >> 7, sv & 127])
                a_d = plsc.load_gather(adst_v, [dv >> 7, dv & 127])
                e = a_s + a_d
                e = jnp.where(e > 0.0, e, 0.2 * e)
                w = jnp.exp(e)
                gid = base + v * 16 + lax.broadcasted_iota(jnp.int32, (16,), 0)
                w = jnp.where(gid < E_REAL, w, 0.0)
                w_v[pl.ds(v * 16, 16)] = w
                idx_adj[pl.ds(v * 16, 16)] = sv + core_off
            pltpu.async_copy(xl_hbm.at[idx_adj], gbuf, sem).wait()

            def scale(g, cc):
                wv = w_v[pl.ds(g * 16, 16)]
                for l in range(16):
                    r = g * 16 + l
                    s = wv[l]
                    for j in range(fv):
                        gbuf[r, pl.ds(j * 16, 16)] = gbuf[r, pl.ds(j * 16, 16)] * s
                return cc
            lax.fori_loop(0, K // 16, scale, 0)
            pltpu.sync_copy(gbuf, out_sh.at[idx_d], add=True)
            pltpu.sync_copy(w_v, den_sh.at[idx_d], add=True)
            return c
        lax.fori_loop(0, CHUNKS, chunk, 0)
        plsc.subcore_barrier()

        pltpu.sync_copy(den_sh.at[pl.ds(row0, ROWS_PER_TILE)], dbuf)

        def recip(j, c):
            dv = dbuf[pl.ds(j * 16, 16)]
            dbuf[pl.ds(j * 16, 16)] = 1.0 / (dv + 1e-16)
            return c
        lax.fori_loop(0, ROWS_PER_TILE // 16, recip, 0)

        for z in range(ROWS_PER_TILE // K):
            pltpu.sync_copy(out_sh.at[pl.ds(row0 + z * K, K)], gbuf)

            def scale2(g, c):
                rv = dbuf[pl.ds(z * K + g * 16, 16)]
                for l in range(16):
                    r = g * 16 + l
                    s = rv[l]
                    for j in range(fv):
                        gbuf[r, pl.ds(j * 16, 16)] = gbuf[r, pl.ds(j * 16, 16)] * s
                return c
            lax.fori_loop(0, K // 16, scale2, 0)
            pltpu.sync_copy(gbuf, out_hbm.at[pl.ds(core_off + row0 + z * K, K)])

    return gat


def _sc_gat_128(*args):
    return _make_sc_gat(128)(*args)


def _sc_gat_64(*args):
    return _make_sc_gat(64)(*args)


def kernel(x, edge_index, batch, W0, b0, W1, as1, ad1, b1,
           W2, as2, ad2, b2, W3, as3, ad3, b3):
    x_pad = jnp.pad(x, ((0, NHAT - N_NODES), (0, 0)))
    batch_col = jnp.pad(batch, (0, NHAT - N_NODES),
                        constant_values=-1).reshape(NHAT, 1)
    loops = jnp.arange(N_NODES, dtype=jnp.int32)
    pad_e = jnp.zeros((E_PAD - E_REAL,), jnp.int32)
    srcs = jnp.concatenate([edge_index[0], loops, pad_e])
    dsts = jnp.concatenate([edge_index[1], loops, pad_e])

    xl0, xl1, asrc, adst = _tc_layer1(x_pad, W0, b0, W1, as1, ad1)
    u = _sc_gat_128(jnp.concatenate([xl0, xl1], axis=0), srcs, dsts, asrc, adst)

    xl0, xl1, asrc, adst = _tc_mid(u, b1, W2, as2, ad2)
    u = _sc_gat_64(jnp.concatenate([xl0, xl1], axis=0), srcs, dsts, asrc, adst)

    xl0, xl1, asrc, adst = _tc_mid(u, b2, W3, as3, ad3)
    u = _sc_gat_64(jnp.concatenate([xl0, xl1], axis=0), srcs, dsts, asrc, adst)

    return _pool(u, b3, batch_col)


# speedup vs baseline: 34.2999x; 2.3773x over previous
import functools

import jax
import jax.numpy as jnp
from jax import lax
from jax.experimental import pallas as pl
from jax.experimental.pallas import tpu as pltpu
from jax.experimental.pallas import tpu_sc as plsc

N_NODES = 10000
NHAT = 10240
MB = 1024
N_BLKS = NHAT // MB
E_REAL = 330000
TILES = 16
K = 128
CHUNKS = 162
E_PAD = TILES * CHUNKS * K
ROWS_PER_TILE = NHAT // TILES
FH = 64
FV = FH // 16


def _q_specs(nq):
    return [pl.BlockSpec((MB, FH), lambda i: (i, 0)) for _ in range(nq)] + [
        pl.BlockSpec((8, 128), lambda i: (i, 0)),
        pl.BlockSpec((8, 128), lambda i: (i, 0)),
    ]


def _q_shapes(nq):
    return [jax.ShapeDtypeStruct((NHAT, FH), jnp.float32) for _ in range(nq)] + [
        jax.ShapeDtypeStruct((NHAT // 128, 128), jnp.float32),
        jax.ShapeDtypeStruct((NHAT // 128, 128), jnp.float32),
    ]


def _emit_outs(xl, avs, avd, out_refs):
    nq = len(out_refs) - 2
    for i in range(nq):
        out_refs[i][...] = xl[:, i * FH:(i + 1) * FH]
    out_refs[nq][...] = jnp.sum(xl * avs, axis=1).reshape(8, 128)
    out_refs[nq + 1][...] = jnp.sum(xl * avd, axis=1).reshape(8, 128)


def _tc1_body(x_ref, w0_ref, b0_ref, w1_ref, avs_ref, avd_ref, *out_refs):
    t = jnp.dot(x_ref[...], w0_ref[...], preferred_element_type=jnp.float32)
    t = t + b0_ref[...]
    xl = jnp.dot(t, w1_ref[...], preferred_element_type=jnp.float32)
    _emit_outs(xl, avs_ref[...], avd_ref[...], out_refs)


def _tc_layer1(x, w0, b0, w1, avs, avd):
    fo = w1.shape[1]
    nq = fo // FH
    return pl.pallas_call(
        _tc1_body,
        grid=(N_BLKS,),
        in_specs=[
            pl.BlockSpec((MB, 128), lambda i: (i, 0)),
            pl.BlockSpec((128, 128), lambda i: (0, 0)),
            pl.BlockSpec((1, 128), lambda i: (0, 0)),
            pl.BlockSpec((128, fo), lambda i: (0, 0)),
            pl.BlockSpec((1, fo), lambda i: (0, 0)),
            pl.BlockSpec((1, fo), lambda i: (0, 0)),
        ],
        out_specs=_q_specs(nq),
        out_shape=_q_shapes(nq),
    )(x, w0, b0.reshape(1, -1), w1, avs.reshape(1, -1), avd.reshape(1, -1))


def _make_tcmid_body(npieces):
    def body(*refs):
        piece_refs = refs[:npieces]
        bp_ref, w_ref, avs_ref, avd_ref = refs[npieces:npieces + 4]
        out_refs = refs[npieces + 4:]
        b = bp_ref[...]
        w = w_ref[...]
        xl = None
        for i in range(npieces):
            h = jnp.maximum(piece_refs[i][...] + b[:, i * FH:(i + 1) * FH], 0.0)
            part = jnp.dot(h, w[i * FH:(i + 1) * FH, :],
                           preferred_element_type=jnp.float32)
            xl = part if xl is None else xl + part
        _emit_outs(xl, avs_ref[...], avd_ref[...], out_refs)
    return body


def _tc_mid(u_list, bp, w, avs, avd):
    fin, fo = w.shape
    nq = fo // FH
    npieces = 2 * len(u_list)
    in_specs = []
    args = []
    for u in u_list:
        in_specs.append(pl.BlockSpec((MB, FH), lambda i: (i, 0)))
        in_specs.append(pl.BlockSpec((MB, FH), lambda i: (i + N_BLKS, 0)))
        args += [u, u]
    in_specs += [
        pl.BlockSpec((1, fin), lambda i: (0, 0)),
        pl.BlockSpec((fin, fo), lambda i: (0, 0)),
        pl.BlockSpec((1, fo), lambda i: (0, 0)),
        pl.BlockSpec((1, fo), lambda i: (0, 0)),
    ]
    args += [bp.reshape(1, -1), w, avs.reshape(1, -1), avd.reshape(1, -1)]
    return pl.pallas_call(
        _make_tcmid_body(npieces),
        grid=(N_BLKS,),
        in_specs=in_specs,
        out_specs=_q_specs(nq),
        out_shape=_q_shapes(nq),
    )(*args)


def _pool_body(u0_ref, u1_ref, b3_ref, batch_ref, out_ref, sums, cnts):
    i = pl.program_id(0)

    @pl.when(i == 0)
    def _():
        sums[...] = jnp.zeros_like(sums)
        cnts[...] = jnp.zeros_like(cnts)

    b = b3_ref[...]
    h = jnp.maximum(jnp.concatenate([u0_ref[...], u1_ref[...]], axis=1) + b, 0.0)
    groups = lax.broadcasted_iota(jnp.int32, (MB, 16), 1)
    oh = (batch_ref[...] == groups).astype(jnp.float32)
    dn = (((0,), (0,)), ((), ()))
    sums[...] += lax.dot_general(oh, h, dn, preferred_element_type=jnp.float32)
    cnts[...] += lax.dot_general(oh, jnp.ones_like(h), dn,
                                 preferred_element_type=jnp.float32)

    @pl.when(i == N_BLKS - 1)
    def _():
        out_ref[...] = sums[...] / jnp.maximum(cnts[...], 1.0)


def _pool(u_flat, b3, batch_col):
    return pl.pallas_call(
        _pool_body,
        grid=(N_BLKS,),
        in_specs=[
            pl.BlockSpec((MB, FH), lambda i: (i, 0)),
            pl.BlockSpec((MB, FH), lambda i: (i + N_BLKS, 0)),
            pl.BlockSpec((1, 128), lambda i: (0, 0)),
            pl.BlockSpec((MB, 1), lambda i: (i, 0)),
        ],
        out_specs=pl.BlockSpec((16, 128), lambda i: (0, 0)),
        out_shape=jax.ShapeDtypeStruct((16, 128), jnp.float32),
        scratch_shapes=[
            pltpu.VMEM((16, 128), jnp.float32),
            pltpu.VMEM((16, 128), jnp.float32),
        ],
    )(u_flat, u_flat, b3.reshape(1, -1), batch_col)


@functools.lru_cache(maxsize=None)
def _make_sc_gat():
    mesh = plsc.VectorSubcoreMesh(core_axis_name="c", subcore_axis_name="s")

    @functools.partial(
        pl.kernel,
        out_type=jax.ShapeDtypeStruct((2 * NHAT, FH), jnp.float32),
        mesh=mesh,
        compiler_params=pltpu.CompilerParams(needs_layout_passes=False,
                                             use_tc_tiling_on_sc=False),
        scratch_types=[
            pltpu.VMEM((NHAT // 128, 128), jnp.float32),
            pltpu.VMEM((NHAT // 128, 128), jnp.float32),
            pltpu.VMEM((6, 2, K), jnp.int32),
            pltpu.VMEM((2, K), jnp.int32),
            pltpu.VMEM((2, K, FH), jnp.float32),
            pltpu.VMEM((2, K, FH), jnp.float32),
            pltpu.VMEM((2, K), jnp.float32),
            pltpu.VMEM((ROWS_PER_TILE,), jnp.float32),
            pltpu.VMEM_SHARED((NHAT, FH), jnp.float32),
            pltpu.VMEM_SHARED((NHAT,), jnp.float32),
        ] + [pltpu.SemaphoreType.DMA] * 12,
    )
    def gat(xl_hbm, eidx_hbm, asrc_hbm, adst_hbm, out_hbm,
            asrc_v, adst_v, sdbuf, sadjb, ibuf, obuf, wbuf, dbuf,
            out_sh, den_sh, sg0, sg1, ss0, ss1, sd0, sd1,
            si0, si1, si2, si3, si4, si5):
        cid = lax.axis_index("c")
        tid = lax.axis_index("s")
        row0 = tid * ROWS_PER_TILE
        core_off = cid * NHAT
        ch0 = tid * CHUNKS
        sg = (sg0, sg1)
        ss = (ss0, ss1)
        sd = (sd0, sd1)
        si = (si0, si1, si2, si3, si4, si5)

        def zrow(r, c):
            for j in range(FV):
                ibuf[0, r, pl.ds(j * 16, 16)] = jnp.zeros((16,), jnp.float32)
            return c
        lax.fori_loop(0, K, zrow, 0)
        for z in range(ROWS_PER_TILE // K):
            pltpu.sync_copy(ibuf.at[0], out_sh.at[pl.ds(row0 + z * K, K)])

        def zden(j, c):
            dbuf[pl.ds(j * 16, 16)] = jnp.zeros((16,), jnp.float32)
            return c
        lax.fori_loop(0, ROWS_PER_TILE // 16, zden, 0)
        pltpu.sync_copy(dbuf, den_sh.at[pl.ds(row0, ROWS_PER_TILE)])

        pltpu.sync_copy(asrc_hbm, asrc_v)
        pltpu.sync_copy(adst_hbm, adst_v)
        for q in range(4):
            pltpu.async_copy(eidx_hbm.at[ch0 + q], sdbuf.at[q], si[q])
        plsc.subcore_barrier()

        for b in range(2):
            pltpu.make_async_copy(eidx_hbm.at[ch0 + b], sdbuf.at[b],
                                  si[b]).wait()
            for v in range(K // 16):
                sadjb[b, pl.ds(v * 16, 16)] = (
                    sdbuf[b, 0, pl.ds(v * 16, 16)] + core_off)
            pltpu.async_copy(xl_hbm.at[sadjb.at[b]], ibuf.at[b], sg[b])

        iot = lax.broadcasted_iota(jnp.int32, (16,), 0)

        def group(g, c):
            for sub in range(6):
                b = sub % 2
                q = sub
                ch = g * 6 + sub
                base = (ch0 + ch) * K
                pltpu.make_async_copy(xl_hbm.at[sadjb.at[b]], ibuf.at[b],
                                      sg[b]).wait()

                @pl.when(ch >= 2)
                def _():
                    pltpu.make_async_copy(obuf.at[b],
                                          out_sh.at[sdbuf.at[q, 1]],
                                          ss[b]).wait()
                    pltpu.make_async_copy(wbuf.at[b],
                                          den_sh.at[sdbuf.at[q, 1]],
                                          sd[b]).wait()

                for v in range(K // 16):
                    sv = sdbuf[q, 0, pl.ds(v * 16, 16)]
                    dv = sdbuf[q, 1, pl.ds(v * 16, 16)]
                    a_s = plsc.load_gather(asrc_v, [sv >> 7, sv & 127])
                    a_d = plsc.load_gather(adst_v, [dv >> 7, dv & 127])
                    e = a_s + a_d
                    e = jnp.where(e > 0.0, e, 0.2 * e)
                    w = jnp.exp(e)
                    w = jnp.where(base + v * 16 + iot < E_REAL, w, 0.0)
                    wbuf[b, pl.ds(v * 16, 16)] = w

                def scale(gg, cc):
                    wv = wbuf[b, pl.ds(gg * 16, 16)]
                    for l in range(16):
                        r = gg * 16 + l
                        s = wv[l]
                        for j in range(FV):
                            obuf[b, r, pl.ds(j * 16, 16)] = (
                                ibuf[b, r, pl.ds(j * 16, 16)] * s)
                    return cc
                lax.fori_loop(0, K // 16, scale, 0)

                pltpu.async_copy(obuf.at[b], out_sh.at[sdbuf.at[q, 1]],
                                 ss[b], add=True)
                pltpu.async_copy(wbuf.at[b], den_sh.at[sdbuf.at[q, 1]],
                                 sd[b], add=True)

                @pl.when(ch + 4 < CHUNKS)
                def _():
                    pltpu.async_copy(eidx_hbm.at[ch0 + ch + 4],
                                     sdbuf.at[(q + 4) % 6], si[(q + 4) % 6])

                @pl.when(ch + 2 < CHUNKS)
                def _():
                    pltpu.make_async_copy(eidx_hbm.at[ch0 + ch + 2],
                                          sdbuf.at[(q + 2) % 6],
                                          si[(q + 2) % 6]).wait()
                    for v in range(K // 16):
                        sadjb[b, pl.ds(v * 16, 16)] = (
                            sdbuf[(q + 2) % 6, 0, pl.ds(v * 16, 16)]
                            + core_off)
                    pltpu.async_copy(xl_hbm.at[sadjb.at[b]], ibuf.at[b],
                                     sg[b])
            return c
        lax.fori_loop(0, CHUNKS // 6, group, 0)

        for b in range(2):
            pltpu.make_async_copy(obuf.at[b], out_sh.at[sdbuf.at[0, 1]],
                                  ss[b]).wait()
            pltpu.make_async_copy(wbuf.at[b], den_sh.at[sdbuf.at[0, 1]],
                                  sd[b]).wait()
        plsc.subcore_barrier()

        pltpu.sync_copy(den_sh.at[pl.ds(row0, ROWS_PER_TILE)], dbuf)

        def recip(j, c):
            dv = dbuf[pl.ds(j * 16, 16)]
            dbuf[pl.ds(j * 16, 16)] = 1.0 / (dv + 1e-16)
            return c
        lax.fori_loop(0, ROWS_PER_TILE // 16, recip, 0)

        for z in range(ROWS_PER_TILE // K):
            pltpu.sync_copy(out_sh.at[pl.ds(row0 + z * K, K)], ibuf.at[0])

            def scale2(g, c):
                rv = dbuf[pl.ds(z * K + g * 16, 16)]
                for l in range(16):
                    r = g * 16 + l
                    s = rv[l]
                    for j in range(FV):
                        ibuf[0, r, pl.ds(j * 16, 16)] = (
                            ibuf[0, r, pl.ds(j * 16, 16)] * s)
                return c
            lax.fori_loop(0, K // 16, scale2, 0)
            pltpu.sync_copy(ibuf.at[0],
                            out_hbm.at[pl.ds(core_off + row0 + z * K, K)])

    return gat


def _sc_gat(xl_flat, eidx, asrc, adst):
    return _make_sc_gat()(xl_flat, eidx, asrc, adst)


def kernel(x, edge_index, batch, W0, b0, W1, as1, ad1, b1,
           W2, as2, ad2, b2, W3, as3, ad3, b3):
    x_pad = jnp.pad(x, ((0, NHAT - N_NODES), (0, 0)))
    batch_col = jnp.pad(batch, (0, NHAT - N_NODES),
                        constant_values=-1).reshape(NHAT, 1)
    loops = jnp.arange(N_NODES, dtype=jnp.int32)
    pad_e = jnp.zeros((E_PAD - E_REAL,), jnp.int32)
    srcs = jnp.concatenate([edge_index[0], loops, pad_e]).reshape(E_PAD // K, K)
    dsts = jnp.concatenate([edge_index[1], loops, pad_e]).reshape(E_PAD // K, K)
    eidx = jnp.stack([srcs, dsts], axis=1)

    q0, q1, q2, q3, asrc, adst = _tc_layer1(x_pad, W0, b0, W1, as1, ad1)
    uA = _sc_gat(jnp.concatenate([q0, q1], axis=0), eidx, asrc, adst)
    uB = _sc_gat(jnp.concatenate([q2, q3], axis=0), eidx, asrc, adst)

    q0, q1, asrc, adst = _tc_mid([uA, uB], b1, W2, as2, ad2)
    u = _sc_gat(jnp.concatenate([q0, q1], axis=0), eidx, asrc, adst)

    q0, q1, asrc, adst = _tc_mid([u], b2, W3, as3, ad3)
    u = _sc_gat(jnp.concatenate([q0, q1], axis=0), eidx, asrc, adst)

    return _pool(u, b3, batch_col)


# speedup vs baseline: 35.9450x; 1.0480x over previous
import functools

import jax
import jax.numpy as jnp
from jax import lax
from jax.experimental import pallas as pl
from jax.experimental.pallas import tpu as pltpu
from jax.experimental.pallas import tpu_sc as plsc

N_NODES = 10000
NHAT = 10240
MB = 1024
N_BLKS = NHAT // MB
E_REAL = 330000
TILES = 16
K = 128
CHUNKS = 162
E_PAD = TILES * CHUNKS * K
ROWS_PER_TILE = NHAT // TILES
FH = 64
FV = FH // 16


def _q_specs(nq):
    return [pl.BlockSpec((MB, FH), lambda i: (i, 0)) for _ in range(nq)] + [
        pl.BlockSpec((8, 128), lambda i: (i, 0)),
        pl.BlockSpec((8, 128), lambda i: (i, 0)),
    ]


def _q_shapes(nq):
    return [jax.ShapeDtypeStruct((NHAT, FH), jnp.float32) for _ in range(nq)] + [
        jax.ShapeDtypeStruct((NHAT // 128, 128), jnp.float32),
        jax.ShapeDtypeStruct((NHAT // 128, 128), jnp.float32),
    ]


def _emit_outs(xl, avs, avd, out_refs):
    nq = len(out_refs) - 2
    for i in range(nq):
        out_refs[i][...] = xl[:, i * FH:(i + 1) * FH]
    out_refs[nq][...] = jnp.sum(xl * avs, axis=1).reshape(8, 128)
    out_refs[nq + 1][...] = jnp.sum(xl * avd, axis=1).reshape(8, 128)


def _tc1_body(x_ref, w0_ref, b0_ref, w1_ref, avs_ref, avd_ref, *out_refs):
    t = jnp.dot(x_ref[...], w0_ref[...], preferred_element_type=jnp.float32)
    t = t + b0_ref[...]
    xl = jnp.dot(t, w1_ref[...], preferred_element_type=jnp.float32)
    _emit_outs(xl, avs_ref[...], avd_ref[...], out_refs)


def _tc_layer1(x, w0, b0, w1, avs, avd):
    fo = w1.shape[1]
    nq = fo // FH
    return pl.pallas_call(
        _tc1_body,
        grid=(N_BLKS,),
        in_specs=[
            pl.BlockSpec((MB, 128), lambda i: (i, 0)),
            pl.BlockSpec((128, 128), lambda i: (0, 0)),
            pl.BlockSpec((1, 128), lambda i: (0, 0)),
            pl.BlockSpec((128, fo), lambda i: (0, 0)),
            pl.BlockSpec((1, fo), lambda i: (0, 0)),
            pl.BlockSpec((1, fo), lambda i: (0, 0)),
        ],
        out_specs=_q_specs(nq),
        out_shape=_q_shapes(nq),
    )(x, w0, b0.reshape(1, -1), w1, avs.reshape(1, -1), avd.reshape(1, -1))


def _make_tcmid_body(npieces):
    def body(*refs):
        piece_refs = refs[:npieces]
        bp_ref, w_ref, avs_ref, avd_ref = refs[npieces:npieces + 4]
        out_refs = refs[npieces + 4:]
        b = bp_ref[...]
        w = w_ref[...]
        xl = None
        for i in range(npieces):
            h = jnp.maximum(piece_refs[i][...] + b[:, i * FH:(i + 1) * FH], 0.0)
            part = jnp.dot(h, w[i * FH:(i + 1) * FH, :],
                           preferred_element_type=jnp.float32)
            xl = part if xl is None else xl + part
        _emit_outs(xl, avs_ref[...], avd_ref[...], out_refs)
    return body


def _tc_mid(u_list, bp, w, avs, avd):
    fin, fo = w.shape
    nq = fo // FH
    npieces = 2 * len(u_list)
    in_specs = []
    args = []
    for u in u_list:
        in_specs.append(pl.BlockSpec((MB, FH), lambda i: (i, 0)))
        in_specs.append(pl.BlockSpec((MB, FH), lambda i: (i + N_BLKS, 0)))
        args += [u, u]
    in_specs += [
        pl.BlockSpec((1, fin), lambda i: (0, 0)),
        pl.BlockSpec((fin, fo), lambda i: (0, 0)),
        pl.BlockSpec((1, fo), lambda i: (0, 0)),
        pl.BlockSpec((1, fo), lambda i: (0, 0)),
    ]
    args += [bp.reshape(1, -1), w, avs.reshape(1, -1), avd.reshape(1, -1)]
    return pl.pallas_call(
        _make_tcmid_body(npieces),
        grid=(N_BLKS,),
        in_specs=in_specs,
        out_specs=_q_specs(nq),
        out_shape=_q_shapes(nq),
    )(*args)


def _pool_body(u0_ref, u1_ref, b3_ref, batch_ref, out_ref, sums, cnts):
    i = pl.program_id(0)

    @pl.when(i == 0)
    def _():
        sums[...] = jnp.zeros_like(sums)
        cnts[...] = jnp.zeros_like(cnts)

    b = b3_ref[...]
    h = jnp.maximum(jnp.concatenate([u0_ref[...], u1_ref[...]], axis=1) + b, 0.0)
    groups = lax.broadcasted_iota(jnp.int32, (MB, 16), 1)
    oh = (batch_ref[...] == groups).astype(jnp.float32)
    dn = (((0,), (0,)), ((), ()))
    sums[...] += lax.dot_general(oh, h, dn, preferred_element_type=jnp.float32)
    cnts[...] += lax.dot_general(oh, jnp.ones_like(h), dn,
                                 preferred_element_type=jnp.float32)

    @pl.when(i == N_BLKS - 1)
    def _():
        out_ref[...] = sums[...] / jnp.maximum(cnts[...], 1.0)


def _pool(u_flat, b3, batch_col):
    return pl.pallas_call(
        _pool_body,
        grid=(N_BLKS,),
        in_specs=[
            pl.BlockSpec((MB, FH), lambda i: (i, 0)),
            pl.BlockSpec((MB, FH), lambda i: (i + N_BLKS, 0)),
            pl.BlockSpec((1, 128), lambda i: (0, 0)),
            pl.BlockSpec((MB, 1), lambda i: (i, 0)),
        ],
        out_specs=pl.BlockSpec((16, 128), lambda i: (0, 0)),
        out_shape=jax.ShapeDtypeStruct((16, 128), jnp.float32),
        scratch_shapes=[
            pltpu.VMEM((16, 128), jnp.float32),
            pltpu.VMEM((16, 128), jnp.float32),
        ],
    )(u_flat, u_flat, b3.reshape(1, -1), batch_col)


@functools.lru_cache(maxsize=None)
def _make_sc_gat():
    mesh = plsc.VectorSubcoreMesh(core_axis_name="c", subcore_axis_name="s")

    @functools.partial(
        pl.kernel,
        out_type=jax.ShapeDtypeStruct((2 * NHAT, FH), jnp.float32),
        mesh=mesh,
        compiler_params=pltpu.CompilerParams(needs_layout_passes=False,
                                             use_tc_tiling_on_sc=False),
        scratch_types=[
            pltpu.VMEM((NHAT // 128, 128), jnp.float32),
            pltpu.VMEM((NHAT // 128, 128), jnp.float32),
            pltpu.VMEM((6, 2, K), jnp.int32),
            pltpu.VMEM((2, K, FH), jnp.float32),
            pltpu.VMEM((2, K, FH), jnp.float32),
            pltpu.VMEM((2, K), jnp.float32),
            pltpu.VMEM((ROWS_PER_TILE,), jnp.float32),
            pltpu.VMEM_SHARED((NHAT, FH), jnp.float32),
            pltpu.VMEM_SHARED((NHAT,), jnp.float32),
        ] + [pltpu.SemaphoreType.DMA] * 12,
    )
    def gat(xlo_hbm, xhi_hbm, eidx_hbm, asrc_hbm, adst_hbm, out_hbm,
            asrc_v, adst_v, sdbuf, ibuf, obuf, wbuf, dbuf,
            out_sh, den_sh, sg0, sg1, ss0, ss1, sd0, sd1,
            si0, si1, si2, si3, si4, si5):
        cid = lax.axis_index("c")
        tid = lax.axis_index("s")
        row0 = tid * ROWS_PER_TILE
        core_off = cid * NHAT
        ch0 = tid * CHUNKS
        sg = (sg0, sg1)

        def issue_gather(qslot, b):
            @pl.when(cid == 0)
            def _():
                pltpu.async_copy(xlo_hbm.at[sdbuf.at[qslot, 0]],
                                 ibuf.at[b], sg[b])

            @pl.when(cid == 1)
            def _():
                pltpu.async_copy(xhi_hbm.at[sdbuf.at[qslot, 0]],
                                 ibuf.at[b], sg[b])

        ss = (ss0, ss1)
        sd = (sd0, sd1)
        si = (si0, si1, si2, si3, si4, si5)

        def zrow(r, c):
            for j in range(FV):
                ibuf[0, r, pl.ds(j * 16, 16)] = jnp.zeros((16,), jnp.float32)
            return c
        lax.fori_loop(0, K, zrow, 0)
        for z in range(ROWS_PER_TILE // K):
            pltpu.sync_copy(ibuf.at[0], out_sh.at[pl.ds(row0 + z * K, K)])

        def zden(j, c):
            dbuf[pl.ds(j * 16, 16)] = jnp.zeros((16,), jnp.float32)
            return c
        lax.fori_loop(0, ROWS_PER_TILE // 16, zden, 0)
        pltpu.sync_copy(dbuf, den_sh.at[pl.ds(row0, ROWS_PER_TILE)])

        pltpu.sync_copy(asrc_hbm, asrc_v)
        pltpu.sync_copy(adst_hbm, adst_v)
        for q in range(4):
            pltpu.async_copy(eidx_hbm.at[ch0 + q], sdbuf.at[q], si[q])
        plsc.subcore_barrier()

        for b in range(2):
            pltpu.make_async_copy(eidx_hbm.at[ch0 + b], sdbuf.at[b],
                                  si[b]).wait()
            issue_gather(b, b)

        iot = lax.broadcasted_iota(jnp.int32, (16,), 0)

        def group(g, c):
            for sub in range(6):
                b = sub % 2
                q = sub
                ch = g * 6 + sub
                base = (ch0 + ch) * K
                pltpu.make_async_copy(xlo_hbm.at[sdbuf.at[q, 0]], ibuf.at[b],
                                      sg[b]).wait()

                @pl.when(ch >= 2)
                def _():
                    pltpu.make_async_copy(obuf.at[b],
                                          out_sh.at[sdbuf.at[q, 1]],
                                          ss[b]).wait()
                    pltpu.make_async_copy(wbuf.at[b],
                                          den_sh.at[sdbuf.at[q, 1]],
                                          sd[b]).wait()

                for v in range(K // 16):
                    sv = sdbuf[q, 0, pl.ds(v * 16, 16)]
                    dv = sdbuf[q, 1, pl.ds(v * 16, 16)]
                    a_s = plsc.load_gather(asrc_v, [sv >> 7, sv & 127])
                    a_d = plsc.load_gather(adst_v, [dv >> 7, dv & 127])
                    e = a_s + a_d
                    e = jnp.where(e > 0.0, e, 0.2 * e)
                    w = jnp.exp(e)
                    w = jnp.where(base + v * 16 + iot < E_REAL, w, 0.0)
                    wbuf[b, pl.ds(v * 16, 16)] = w

                def scale(gg, cc):
                    wv = wbuf[b, pl.ds(gg * 16, 16)]
                    for l in range(16):
                        r = gg * 16 + l
                        s = wv[l]
                        for j in range(FV):
                            obuf[b, r, pl.ds(j * 16, 16)] = (
                                ibuf[b, r, pl.ds(j * 16, 16)] * s)
                    return cc
                lax.fori_loop(0, K // 16, scale, 0, unroll=2)

                pltpu.async_copy(obuf.at[b], out_sh.at[sdbuf.at[q, 1]],
                                 ss[b], add=True)
                pltpu.async_copy(wbuf.at[b], den_sh.at[sdbuf.at[q, 1]],
                                 sd[b], add=True)

                @pl.when(ch + 4 < CHUNKS)
                def _():
                    pltpu.async_copy(eidx_hbm.at[ch0 + ch + 4],
                                     sdbuf.at[(q + 4) % 6], si[(q + 4) % 6])

                @pl.when(ch + 2 < CHUNKS)
                def _():
                    pltpu.make_async_copy(eidx_hbm.at[ch0 + ch + 2],
                                          sdbuf.at[(q + 2) % 6],
                                          si[(q + 2) % 6]).wait()
                    issue_gather((q + 2) % 6, b)
            return c
        lax.fori_loop(0, CHUNKS // 6, group, 0)

        for b in range(2):
            pltpu.make_async_copy(obuf.at[b], out_sh.at[sdbuf.at[0, 1]],
                                  ss[b]).wait()
            pltpu.make_async_copy(wbuf.at[b], den_sh.at[sdbuf.at[0, 1]],
                                  sd[b]).wait()
        plsc.subcore_barrier()

        pltpu.sync_copy(den_sh.at[pl.ds(row0, ROWS_PER_TILE)], dbuf)

        def recip(j, c):
            dv = dbuf[pl.ds(j * 16, 16)]
            dbuf[pl.ds(j * 16, 16)] = 1.0 / (dv + 1e-16)
            return c
        lax.fori_loop(0, ROWS_PER_TILE // 16, recip, 0)

        for z in range(ROWS_PER_TILE // K):
            pltpu.sync_copy(out_sh.at[pl.ds(row0 + z * K, K)], ibuf.at[0])

            def scale2(g, c):
                rv = dbuf[pl.ds(z * K + g * 16, 16)]
                for l in range(16):
                    r = g * 16 + l
                    s = rv[l]
                    for j in range(FV):
                        ibuf[0, r, pl.ds(j * 16, 16)] = (
                            ibuf[0, r, pl.ds(j * 16, 16)] * s)
                return c
            lax.fori_loop(0, K // 16, scale2, 0)
            pltpu.sync_copy(ibuf.at[0],
                            out_hbm.at[pl.ds(core_off + row0 + z * K, K)])

    return gat


def _sc_gat(xlo, xhi, eidx, asrc, adst):
    return _make_sc_gat()(xlo, xhi, eidx, asrc, adst)


def kernel(x, edge_index, batch, W0, b0, W1, as1, ad1, b1,
           W2, as2, ad2, b2, W3, as3, ad3, b3):
    x_pad = jnp.pad(x, ((0, NHAT - N_NODES), (0, 0)))
    batch_col = jnp.pad(batch, (0, NHAT - N_NODES),
                        constant_values=-1).reshape(NHAT, 1)
    loops = jnp.arange(N_NODES, dtype=jnp.int32)
    pad_e = jnp.zeros((E_PAD - E_REAL,), jnp.int32)
    srcs = jnp.concatenate([edge_index[0], loops, pad_e]).reshape(E_PAD // K, K)
    dsts = jnp.concatenate([edge_index[1], loops, pad_e]).reshape(E_PAD // K, K)
    eidx = jnp.stack([srcs, dsts], axis=1)

    q0, q1, q2, q3, asrc, adst = _tc_layer1(x_pad, W0, b0, W1, as1, ad1)
    uA = _sc_gat(q0, q1, eidx, asrc, adst)
    uB = _sc_gat(q2, q3, eidx, asrc, adst)

    q0, q1, asrc, adst = _tc_mid([uA, uB], b1, W2, as2, ad2)
    u = _sc_gat(q0, q1, eidx, asrc, adst)

    q0, q1, asrc, adst = _tc_mid([u], b2, W3, as3, ad3)
    u = _sc_gat(q0, q1, eidx, asrc, adst)

    return _pool(u, b3, batch_col)


# speedup vs baseline: 37.3614x; 1.0394x over previous
import functools

import jax
import jax.numpy as jnp
from jax import lax
from jax.experimental import pallas as pl
from jax.experimental.pallas import tpu as pltpu
from jax.experimental.pallas import tpu_sc as plsc

N_NODES = 10000
NHAT = 10240
MB = 1024
N_BLKS = NHAT // MB
E_REAL = 330000
TILES = 16
K = 128
CHUNKS = 162
E_PAD = TILES * CHUNKS * K
ROWS_PER_TILE = NHAT // TILES
FH = 64
FV = FH // 16


def _q_specs(nq):
    return [pl.BlockSpec((MB, FH), lambda i: (i, 0)) for _ in range(nq)] + [
        pl.BlockSpec((8, 128), lambda i: (i, 0)),
        pl.BlockSpec((8, 128), lambda i: (i, 0)),
    ]


def _q_shapes(nq):
    return [jax.ShapeDtypeStruct((NHAT, FH), jnp.float32) for _ in range(nq)] + [
        jax.ShapeDtypeStruct((NHAT // 128, 128), jnp.float32),
        jax.ShapeDtypeStruct((NHAT // 128, 128), jnp.float32),
    ]


def _emit_outs(xl, avs, avd, out_refs):
    nq = len(out_refs) - 2
    for i in range(nq):
        out_refs[i][...] = xl[:, i * FH:(i + 1) * FH]
    out_refs[nq][...] = jnp.sum(xl * avs, axis=1).reshape(8, 128)
    out_refs[nq + 1][...] = jnp.sum(xl * avd, axis=1).reshape(8, 128)


def _tc1_body(x_ref, w0_ref, b0_ref, w1_ref, avs_ref, avd_ref, *out_refs):
    t = jnp.dot(x_ref[...], w0_ref[...], preferred_element_type=jnp.float32)
    t = t + b0_ref[...]
    xl = jnp.dot(t, w1_ref[...], preferred_element_type=jnp.float32)
    _emit_outs(xl, avs_ref[...], avd_ref[...], out_refs)


def _tc_layer1(x, w0, b0, w1, avs, avd):
    fo = w1.shape[1]
    nq = fo // FH
    return pl.pallas_call(
        _tc1_body,
        grid=(N_BLKS,),
        in_specs=[
            pl.BlockSpec((MB, 128), lambda i: (i, 0)),
            pl.BlockSpec((128, 128), lambda i: (0, 0)),
            pl.BlockSpec((1, 128), lambda i: (0, 0)),
            pl.BlockSpec((128, fo), lambda i: (0, 0)),
            pl.BlockSpec((1, fo), lambda i: (0, 0)),
            pl.BlockSpec((1, fo), lambda i: (0, 0)),
        ],
        out_specs=_q_specs(nq),
        out_shape=_q_shapes(nq),
    )(x, w0, b0.reshape(1, -1), w1, avs.reshape(1, -1), avd.reshape(1, -1))


def _make_tcmid_body(npieces):
    def body(*refs):
        piece_refs = refs[:npieces]
        bp_ref, w_ref, avs_ref, avd_ref = refs[npieces:npieces + 4]
        out_refs = refs[npieces + 4:]
        b = bp_ref[...]
        w = w_ref[...]
        xl = None
        for i in range(npieces):
            h = jnp.maximum(piece_refs[i][...] + b[:, i * FH:(i + 1) * FH], 0.0)
            part = jnp.dot(h, w[i * FH:(i + 1) * FH, :],
                           preferred_element_type=jnp.float32)
            xl = part if xl is None else xl + part
        _emit_outs(xl, avs_ref[...], avd_ref[...], out_refs)
    return body


def _tc_mid(u_list, bp, w, avs, avd):
    fin, fo = w.shape
    nq = fo // FH
    npieces = 2 * len(u_list)
    in_specs = []
    args = []
    for u in u_list:
        in_specs.append(pl.BlockSpec((MB, FH), lambda i: (i, 0)))
        in_specs.append(pl.BlockSpec((MB, FH), lambda i: (i + N_BLKS, 0)))
        args += [u, u]
    in_specs += [
        pl.BlockSpec((1, fin), lambda i: (0, 0)),
        pl.BlockSpec((fin, fo), lambda i: (0, 0)),
        pl.BlockSpec((1, fo), lambda i: (0, 0)),
        pl.BlockSpec((1, fo), lambda i: (0, 0)),
    ]
    args += [bp.reshape(1, -1), w, avs.reshape(1, -1), avd.reshape(1, -1)]
    return pl.pallas_call(
        _make_tcmid_body(npieces),
        grid=(N_BLKS,),
        in_specs=in_specs,
        out_specs=_q_specs(nq),
        out_shape=_q_shapes(nq),
    )(*args)


def _pool_body(u0_ref, u1_ref, b3_ref, batch_ref, out_ref, sums, cnts):
    i = pl.program_id(0)

    @pl.when(i == 0)
    def _():
        sums[...] = jnp.zeros_like(sums)
        cnts[...] = jnp.zeros_like(cnts)

    b = b3_ref[...]
    h = jnp.maximum(jnp.concatenate([u0_ref[...], u1_ref[...]], axis=1) + b, 0.0)
    groups = lax.broadcasted_iota(jnp.int32, (MB, 16), 1)
    oh = (batch_ref[...] == groups).astype(jnp.float32)
    dn = (((0,), (0,)), ((), ()))
    sums[...] += lax.dot_general(oh, h, dn, preferred_element_type=jnp.float32)
    cnts[...] += lax.dot_general(oh, jnp.ones_like(h), dn,
                                 preferred_element_type=jnp.float32)

    @pl.when(i == N_BLKS - 1)
    def _():
        out_ref[...] = sums[...] / jnp.maximum(cnts[...], 1.0)


def _pool(u_flat, b3, batch_col):
    return pl.pallas_call(
        _pool_body,
        grid=(N_BLKS,),
        in_specs=[
            pl.BlockSpec((MB, FH), lambda i: (i, 0)),
            pl.BlockSpec((MB, FH), lambda i: (i + N_BLKS, 0)),
            pl.BlockSpec((1, 128), lambda i: (0, 0)),
            pl.BlockSpec((MB, 1), lambda i: (i, 0)),
        ],
        out_specs=pl.BlockSpec((16, 128), lambda i: (0, 0)),
        out_shape=jax.ShapeDtypeStruct((16, 128), jnp.float32),
        scratch_shapes=[
            pltpu.VMEM((16, 128), jnp.float32),
            pltpu.VMEM((16, 128), jnp.float32),
        ],
    )(u_flat, u_flat, b3.reshape(1, -1), batch_col)


@functools.lru_cache(maxsize=None)
def _make_sc_gat():
    mesh = plsc.VectorSubcoreMesh(core_axis_name="c", subcore_axis_name="s")

    @functools.partial(
        pl.kernel,
        out_type=jax.ShapeDtypeStruct((2 * NHAT, FH), jnp.float32),
        mesh=mesh,
        compiler_params=pltpu.CompilerParams(needs_layout_passes=False,
                                             use_tc_tiling_on_sc=False),
        scratch_types=[
            pltpu.VMEM((NHAT // 128, 128), jnp.float32),
            pltpu.VMEM((NHAT // 128, 128), jnp.float32),
            pltpu.VMEM((6, 2, K), jnp.int32),
            pltpu.VMEM((2, K, FH), jnp.float32),
            pltpu.VMEM((2, K, FH), jnp.float32),
            pltpu.VMEM((2, K), jnp.float32),
            pltpu.VMEM((ROWS_PER_TILE,), jnp.float32),
            pltpu.VMEM_SHARED((NHAT, FH), jnp.float32),
            pltpu.VMEM_SHARED((NHAT,), jnp.float32),
        ] + [pltpu.SemaphoreType.DMA] * 12,
    )
    def gat(xlo_hbm, xhi_hbm, eidx_hbm, asrc_hbm, adst_hbm, out_hbm,
            asrc_v, adst_v, sdbuf, ibuf, obuf, wbuf, dbuf,
            out_sh, den_sh, sg0, sg1, ss0, ss1, sd0, sd1,
            si0, si1, si2, si3, si4, si5):
        cid = lax.axis_index("c")
        tid = lax.axis_index("s")
        row0 = tid * ROWS_PER_TILE
        core_off = cid * NHAT
        ch0 = tid * CHUNKS
        sg = (sg0, sg1)

        def issue_gather(qslot, b):
            @pl.when(cid == 0)
            def _():
                pltpu.async_copy(xlo_hbm.at[sdbuf.at[qslot, 0]],
                                 ibuf.at[b], sg[b])

            @pl.when(cid == 1)
            def _():
                pltpu.async_copy(xhi_hbm.at[sdbuf.at[qslot, 0]],
                                 ibuf.at[b], sg[b])

        ss = (ss0, ss1)
        sd = (sd0, sd1)
        si = (si0, si1, si2, si3, si4, si5)

        def zrow(r, c):
            for j in range(FV):
                ibuf[0, r, pl.ds(j * 16, 16)] = jnp.zeros((16,), jnp.float32)
            return c
        lax.fori_loop(0, K, zrow, 0)
        for z in range(ROWS_PER_TILE // K):
            pltpu.sync_copy(ibuf.at[0], out_sh.at[pl.ds(row0 + z * K, K)])

        def zden(j, c):
            dbuf[pl.ds(j * 16, 16)] = jnp.zeros((16,), jnp.float32)
            return c
        lax.fori_loop(0, ROWS_PER_TILE // 16, zden, 0)
        pltpu.sync_copy(dbuf, den_sh.at[pl.ds(row0, ROWS_PER_TILE)])

        pltpu.sync_copy(asrc_hbm, asrc_v)
        pltpu.sync_copy(adst_hbm, adst_v)
        for q in range(4):
            pltpu.async_copy(eidx_hbm.at[ch0 + q], sdbuf.at[q], si[q])
        plsc.subcore_barrier()

        for b in range(2):
            pltpu.make_async_copy(eidx_hbm.at[ch0 + b], sdbuf.at[b],
                                  si[b]).wait()
            issue_gather(b, b)

        iot = lax.broadcasted_iota(jnp.int32, (16,), 0)

        def group(g, c):
            for sub in range(6):
                b = sub % 2
                q = sub
                ch = g * 6 + sub
                base = (ch0 + ch) * K

                @pl.when(ch >= 2)
                def _():
                    pltpu.make_async_copy(obuf.at[b],
                                          out_sh.at[sdbuf.at[q, 1]],
                                          ss[b]).wait()
                    pltpu.make_async_copy(wbuf.at[b],
                                          den_sh.at[sdbuf.at[q, 1]],
                                          sd[b]).wait()

                for v in range(K // 16):
                    sv = sdbuf[q, 0, pl.ds(v * 16, 16)]
                    dv = sdbuf[q, 1, pl.ds(v * 16, 16)]
                    a_s = plsc.load_gather(asrc_v, [sv >> 7, sv & 127])
                    a_d = plsc.load_gather(adst_v, [dv >> 7, dv & 127])
                    e = a_s + a_d
                    e = jnp.where(e > 0.0, e, 0.2 * e)
                    w = jnp.exp(e)
                    w = jnp.where(base + v * 16 + iot < E_REAL, w, 0.0)
                    wbuf[b, pl.ds(v * 16, 16)] = w

                pltpu.make_async_copy(xlo_hbm.at[sdbuf.at[q, 0]], ibuf.at[b],
                                      sg[b]).wait()

                def scale(gg, cc):
                    wv = wbuf[b, pl.ds(gg * 16, 16)]
                    for l in range(16):
                        r = gg * 16 + l
                        s = wv[l]
                        for j in range(FV):
                            obuf[b, r, pl.ds(j * 16, 16)] = (
                                ibuf[b, r, pl.ds(j * 16, 16)] * s)
                    return cc
                lax.fori_loop(0, K // 16, scale, 0, unroll=2)

                pltpu.async_copy(obuf.at[b], out_sh.at[sdbuf.at[q, 1]],
                                 ss[b], add=True)
                pltpu.async_copy(wbuf.at[b], den_sh.at[sdbuf.at[q, 1]],
                                 sd[b], add=True)

                @pl.when(ch + 4 < CHUNKS)
                def _():
                    pltpu.async_copy(eidx_hbm.at[ch0 + ch + 4],
                                     sdbuf.at[(q + 4) % 6], si[(q + 4) % 6])

                @pl.when(ch + 2 < CHUNKS)
                def _():
                    pltpu.make_async_copy(eidx_hbm.at[ch0 + ch + 2],
                                          sdbuf.at[(q + 2) % 6],
                                          si[(q + 2) % 6]).wait()
                    issue_gather((q + 2) % 6, b)
            return c
        lax.fori_loop(0, CHUNKS // 6, group, 0)

        for b in range(2):
            pltpu.make_async_copy(obuf.at[b], out_sh.at[sdbuf.at[0, 1]],
                                  ss[b]).wait()
            pltpu.make_async_copy(wbuf.at[b], den_sh.at[sdbuf.at[0, 1]],
                                  sd[b]).wait()
        plsc.subcore_barrier()

        pltpu.sync_copy(den_sh.at[pl.ds(row0, ROWS_PER_TILE)], dbuf)

        def recip(j, c):
            dv = dbuf[pl.ds(j * 16, 16)]
            dbuf[pl.ds(j * 16, 16)] = 1.0 / (dv + 1e-16)
            return c
        lax.fori_loop(0, ROWS_PER_TILE // 16, recip, 0)

        for z in range(ROWS_PER_TILE // K):
            pltpu.sync_copy(out_sh.at[pl.ds(row0 + z * K, K)], ibuf.at[0])

            def scale2(g, c):
                rv = dbuf[pl.ds(z * K + g * 16, 16)]
                for l in range(16):
                    r = g * 16 + l
                    s = rv[l]
                    for j in range(FV):
                        ibuf[0, r, pl.ds(j * 16, 16)] = (
                            ibuf[0, r, pl.ds(j * 16, 16)] * s)
                return c
            lax.fori_loop(0, K // 16, scale2, 0)
            pltpu.sync_copy(ibuf.at[0],
                            out_hbm.at[pl.ds(core_off + row0 + z * K, K)])

    return gat


def _sc_gat(xlo, xhi, eidx, asrc, adst):
    return _make_sc_gat()(xlo, xhi, eidx, asrc, adst)


def kernel(x, edge_index, batch, W0, b0, W1, as1, ad1, b1,
           W2, as2, ad2, b2, W3, as3, ad3, b3):
    x_pad = jnp.pad(x, ((0, NHAT - N_NODES), (0, 0)))
    batch_col = jnp.pad(batch, (0, NHAT - N_NODES),
                        constant_values=-1).reshape(NHAT, 1)
    loops = jnp.arange(N_NODES, dtype=jnp.int32)
    pad_e = jnp.zeros((E_PAD - E_REAL,), jnp.int32)
    srcs = jnp.concatenate([edge_index[0], loops, pad_e]).reshape(E_PAD // K, K)
    dsts = jnp.concatenate([edge_index[1], loops, pad_e]).reshape(E_PAD // K, K)
    eidx = jnp.stack([srcs, dsts], axis=1)

    q0, q1, q2, q3, asrc, adst = _tc_layer1(x_pad, W0, b0, W1, as1, ad1)
    uA = _sc_gat(q0, q1, eidx, asrc, adst)
    uB = _sc_gat(q2, q3, eidx, asrc, adst)

    q0, q1, asrc, adst = _tc_mid([uA, uB], b1, W2, as2, ad2)
    u = _sc_gat(q0, q1, eidx, asrc, adst)

    q0, q1, asrc, adst = _tc_mid([u], b2, W3, as3, ad3)
    u = _sc_gat(q0, q1, eidx, asrc, adst)

    return _pool(u, b3, batch_col)
